# baseline (speedup 1.0000x reference)
import functools

import jax
import jax.numpy as jnp
from jax import lax
from jax.experimental import pallas as pl
from jax.experimental.pallas import tpu as pltpu
from jax.experimental.pallas import tpu_sc as plsc

B = 16384
D = 16
NC = 2
NS = 16
NW = NC * NS
B_PER_W = B // NW
CHUNK = 128
N_CHUNKS = B_PER_W // CHUNK

_MESH = plsc.VectorSubcoreMesh(
    core_axis_name="c", subcore_axis_name="s", num_cores=NC, num_subcores=NS
)


@functools.partial(
    pl.kernel,
    out_type=(
        jax.ShapeDtypeStruct((B, D), jnp.float32),
        jax.ShapeDtypeStruct((B, D), jnp.float32),
        jax.ShapeDtypeStruct((B, D), jnp.float32),
    ),
    mesh=_MESH,
    compiler_params=pltpu.CompilerParams(use_tc_tiling_on_sc=False),
    scratch_types=[
        pltpu.VMEM((B_PER_W,), jnp.int32),
        pltpu.VMEM((B_PER_W,), jnp.int32),
        pltpu.VMEM((B_PER_W,), jnp.int32),
        pltpu.VMEM((B_PER_W, D), jnp.float32),
        pltpu.VMEM((B_PER_W, D), jnp.float32),
        pltpu.VMEM((B_PER_W, D), jnp.float32),
        pltpu.SemaphoreType.DMA,
    ],
)
def _sc_gather(
    skill_t, loc_t, role_t, s_idx, l_idx, r_idx,
    out_s, out_l, out_r,
    idx_s, idx_l, idx_r, rows_s, rows_l, rows_r, sem,
):
    wid = lax.axis_index("s") * NC + lax.axis_index("c")
    base = wid * B_PER_W
    pltpu.sync_copy(s_idx.at[pl.ds(base, B_PER_W)], idx_s)
    pltpu.sync_copy(l_idx.at[pl.ds(base, B_PER_W)], idx_l)
    pltpu.sync_copy(r_idx.at[pl.ds(base, B_PER_W)], idx_r)
    copies = []
    for tab, idx_v, rows_v in (
        (skill_t, idx_s, rows_s),
        (loc_t, idx_l, rows_l),
        (role_t, idx_r, rows_r),
    ):
        for g in range(N_CHUNKS):
            sl = pl.ds(g * CHUNK, CHUNK)
            copies.append(pltpu.async_copy(tab.at[idx_v.at[sl]], rows_v.at[sl], sem))
    for c in copies:
        c.wait()
    pltpu.sync_copy(rows_s, out_s.at[pl.ds(base, B_PER_W)])
    pltpu.sync_copy(rows_l, out_l.at[pl.ds(base, B_PER_W)])
    pltpu.sync_copy(rows_r, out_r.at[pl.ds(base, B_PER_W)])


BLK = 2048


def _mlp_body(s_ref, l_ref, r_ref, w1s_ref, w1l_ref, w1r_ref, b1_ref,
              w2_ref, b2_ref, w3_ref, b3_ref, out_ref):
    h = (
        jnp.dot(s_ref[...], w1s_ref[...], preferred_element_type=jnp.float32)
        + jnp.dot(l_ref[...], w1l_ref[...], preferred_element_type=jnp.float32)
        + jnp.dot(r_ref[...], w1r_ref[...], preferred_element_type=jnp.float32)
        + b1_ref[...]
    )
    h = jnp.maximum(h, 0.0)
    h2 = jnp.dot(h, w2_ref[...], preferred_element_type=jnp.float32) + b2_ref[...]
    h2 = jnp.maximum(h2, 0.0)
    logit = jnp.sum(h2 * w3_ref[...], axis=1) + b3_ref[0, 0]
    out_ref[...] = jax.nn.sigmoid(logit)


_mlp = pl.pallas_call(
    _mlp_body,
    grid=(B // BLK,),
    in_specs=[
        pl.BlockSpec((BLK, D), lambda i: (i, 0)),
        pl.BlockSpec((BLK, D), lambda i: (i, 0)),
        pl.BlockSpec((BLK, D), lambda i: (i, 0)),
        pl.BlockSpec((D, 64), lambda i: (0, 0)),
        pl.BlockSpec((D, 64), lambda i: (0, 0)),
        pl.BlockSpec((D, 64), lambda i: (0, 0)),
        pl.BlockSpec((1, 64), lambda i: (0, 0)),
        pl.BlockSpec((64, 32), lambda i: (0, 0)),
        pl.BlockSpec((1, 32), lambda i: (0, 0)),
        pl.BlockSpec((1, 32), lambda i: (0, 0)),
        pl.BlockSpec((1, 1), lambda i: (0, 0), memory_space=pltpu.SMEM),
    ],
    out_specs=pl.BlockSpec((BLK,), lambda i: (i,)),
    out_shape=jax.ShapeDtypeStruct((B,), jnp.float32),
)


def kernel(skill_idx, location_idx, role_idx, skill_table, location_table,
           role_table, W1, b1, W2, b2, W3, b3):
    s_idx = skill_idx.astype(jnp.int32)
    l_idx = location_idx.astype(jnp.int32)
    r_idx = role_idx.astype(jnp.int32)
    es, el, er = _sc_gather(skill_table, location_table, role_table,
                            s_idx, l_idx, r_idx)
    return _mlp(
        es, el, er,
        W1[0:D], W1[D:2 * D], W1[2 * D:3 * D],
        b1.reshape(1, 64), W2, b2.reshape(1, 32),
        W3.reshape(1, 32), b3.reshape(1, 1),
    )
